# Initial kernel scaffold; baseline (speedup 1.0000x reference)
import jax
import jax.numpy as jnp
from jax.experimental import pallas as pl


def kernel(A1_tensor, adj_indices, adj_values, batch_idx, n, Lin1, Lin1_bias, W1, b1, W2, b2, W3, b3, weight2, bias2, conv1_w, conv1_b, conv2_w, conv2_b, classifier, classifier_bias):
    raise NotImplementedError("write your pallas kernel here")


# speedup vs baseline: 4.7929x; 4.7929x over previous
import functools

import jax
import jax.numpy as jnp
from jax import lax
from jax.experimental import pallas as pl
from jax.experimental.pallas import tpu as pltpu
from jax.experimental.pallas import tpu_sc as plsc

NN = 10000
EE = 320000
DD = 128
LL = 128
BB = 512
NCLS = 16

NWORK = 32
EPW = EE // NWORK
CK = 80
NCH = EPW // CK
NPAD = NN + 80
NBLK = NPAD // CK

_mesh = plsc.VectorSubcoreMesh(core_axis_name="c", subcore_axis_name="s")


def _spmm_body(src_hbm, dst_hbm, vals_hbm, h_hbm, out_hbm,
               src_v, dst_v, buf_a, buf_b, val_a, val_b, dstw,
               sem_a, sem_b, sem_va, sem_vb, acc):
    c = lax.axis_index("c")
    s = lax.axis_index("s")
    w = c * 16 + s
    ebase = w * EPW

    pltpu.sync_copy(src_hbm.at[pl.ds(ebase, EPW)], src_v)
    pltpu.sync_copy(dst_hbm.at[w, 0], dst_v)

    zero16 = jnp.zeros((16,), jnp.float32)

    def zrow(i, carry):
        for g in range(8):
            buf_a[i, pl.ds(g * 16, 16)] = zero16
        return carry

    lax.fori_loop(0, CK, zrow, 0)
    for t in range(8):
        blk = s + 16 * t

        @pl.when(blk < NBLK)
        def _():
            pltpu.sync_copy(buf_a, acc.at[pl.ds(blk * CK, CK)])
    plsc.subcore_barrier()

    def gather(ch, buf, sem, valb, semv):
        pltpu.async_copy(h_hbm.at[src_v.at[pl.ds(ch * CK, CK)]], buf, sem)
        pltpu.async_copy(vals_hbm.at[pl.ds(ebase + ch * CK, CK)], valb, semv)

    def wait(buf, sem, valb, semv):
        pltpu.make_async_copy(h_hbm.at[pl.ds(0, CK)], buf, sem).wait()
        pltpu.make_async_copy(vals_hbm.at[pl.ds(0, CK)], valb, semv).wait()

    iota16 = lax.broadcasted_iota(jnp.int32, (16,), 0)
    zeros8 = tuple(jnp.zeros((16,), jnp.float32) for _ in range(8))

    def walk_chunk(ch, buf, valb, carry):
        def group(jj, rs):
            e0 = jj * 16
            p0 = 8 + ch * CK + e0
            dv = dst_v[pl.ds(p0, 16)]
            sv_v = valb[pl.ds(e0, 16)]
            nxt = dst_v[pl.ds(p0 + 1, 16)]
            prevv = dst_v[pl.ds(p0 - 1, 16)]
            lastm = jnp.minimum(jnp.abs(dv - nxt), 1)
            keepf = (1 - jnp.minimum(jnp.abs(dv - prevv), 1)).astype(
                jnp.float32)
            dummy_v = NN + ((w * CK + e0 + iota16) & 63)
            dstw[0, pl.ds(e0, 16)] = dv * lastm + dummy_v * (1 - lastm)
            for i in range(16):
                kf = keepf[i]
                sv = sv_v[i]
                rs = tuple(
                    rs[g] * kf + buf[e0 + i, pl.ds(g * 16, 16)] * sv
                    for g in range(8))
                for g in range(8):
                    buf[e0 + i, pl.ds(g * 16, 16)] = rs[g]
            return rs

        carry = lax.fori_loop(0, CK // 16, group, carry)
        pltpu.sync_copy(buf, acc.at[dstw.at[0]], add=True)
        return carry

    gather(0, buf_a, sem_a, val_a, sem_va)
    gather(1, buf_b, sem_b, val_b, sem_vb)

    def pair(i, carry):
        ch0 = 2 * i
        wait(buf_a, sem_a, val_a, sem_va)
        carry = walk_chunk(ch0, buf_a, val_a, carry)
        gather(ch0 + 2, buf_a, sem_a, val_a, sem_va)
        wait(buf_b, sem_b, val_b, sem_vb)
        carry = walk_chunk(ch0 + 1, buf_b, val_b, carry)

        @pl.when(i < 61)
        def _():
            gather(ch0 + 3, buf_b, sem_b, val_b, sem_vb)

        return carry

    carry = lax.fori_loop(0, 62, pair, zeros8)
    wait(buf_a, sem_a, val_a, sem_va)
    walk_chunk(NCH - 1, buf_a, val_a, carry)
    plsc.subcore_barrier()

    for t in range(8):
        blk = s + 16 * t

        @pl.when(blk < NBLK)
        def _():
            pltpu.sync_copy(acc.at[pl.ds(blk * CK, CK)],
                            out_hbm.at[c, pl.ds(blk * CK, CK)])


_spmm_call = pl.kernel(
    _spmm_body,
    out_type=jax.ShapeDtypeStruct((2, NPAD, DD), jnp.float32),
    mesh=_mesh,
    scratch_types=[
        pltpu.VMEM((EPW,), jnp.int32),
        pltpu.VMEM((EPW + 24,), jnp.int32),
        pltpu.VMEM((CK, DD), jnp.float32),
        pltpu.VMEM((CK, DD), jnp.float32),
        pltpu.VMEM((CK,), jnp.float32),
        pltpu.VMEM((CK,), jnp.float32),
        pltpu.VMEM((1, CK), jnp.int32),
        pltpu.SemaphoreType.DMA,
        pltpu.SemaphoreType.DMA,
        pltpu.SemaphoreType.DMA,
        pltpu.SemaphoreType.DMA,
        pltpu.VMEM_SHARED((NPAD, DD), jnp.float32),
    ],
)


def _gred_body(p_hbm, q_hbm, bi_hbm, w1_hbm, w2_hbm, out1_hbm, out2_hbm,
               idx_v, rows, w1_v, w2_v, stage, sem):
    c = lax.axis_index("c")
    s = lax.axis_index("s")
    w = c * 16 + s

    pltpu.sync_copy(w1_hbm, w1_v)
    pltpu.sync_copy(w2_hbm, w2_v)

    for tab, wv, outp in ((p_hbm, w1_v, out1_hbm), (q_hbm, w2_v, out2_hbm)):
        def task(t, carry):
            b = w * 16 + t
            pltpu.sync_copy(bi_hbm.at[b], idx_v)
            pltpu.async_copy(tab.at[idx_v.at[0]], rows, sem).wait()

            def red(jj, accs):
                e0 = jj * 16
                vv = wv[pl.ds(e0, 16)]
                for i in range(16):
                    sv = vv[i]
                    accs = tuple(
                        accs[g] + rows[e0 + i, pl.ds(g * 16, 16)] * sv
                        for g in range(8))
                return accs

            accs = lax.fori_loop(
                0, LL // 16, red,
                tuple(jnp.zeros((16,), jnp.float32) for _ in range(8)))
            for g in range(8):
                stage[0, pl.ds(g * 16, 16)] = accs[g]
            pltpu.sync_copy(stage, outp.at[b])
            return carry

        lax.fori_loop(0, BB // NWORK, task, 0)


_gred_call = pl.kernel(
    _gred_body,
    out_type=(jax.ShapeDtypeStruct((BB, 1, LL), jnp.float32),
              jax.ShapeDtypeStruct((BB, 1, LL), jnp.float32)),
    mesh=_mesh,
    scratch_types=[
        pltpu.VMEM((1, LL), jnp.int32),
        pltpu.VMEM((LL, DD), jnp.float32),
        pltpu.VMEM((LL,), jnp.float32),
        pltpu.VMEM((LL,), jnp.float32),
        pltpu.VMEM((1, LL), jnp.float32),
        pltpu.SemaphoreType.DMA,
    ],
)


_RB = 1000
_GRID = NN // _RB

_row_spec = pl.BlockSpec((_RB, DD), lambda i: (i, 0))
_col_spec = pl.BlockSpec((_RB, 1), lambda i: (i, 0))
_w_spec = pl.BlockSpec((DD, DD), lambda i: (0, 0))
_b_spec = pl.BlockSpec((1, DD), lambda i: (0, 0))


def _bmm(x, w):
    return jnp.dot(x.astype(jnp.bfloat16), w.astype(jnp.bfloat16),
                   preferred_element_type=jnp.float32)


def _round_bf16(x):
    return lax.optimization_barrier(x.astype(jnp.bfloat16)).astype(
        jnp.float32)


def _stage1_body(a1_ref, n_ref, lin1_ref, lb_ref, w1_ref, b1_ref, dw_ref,
                 b2_ref, o_a1, o_h1, o_q):
    x = a1_ref[...]
    a0 = _bmm(x, lin1_ref[...]) + lb_ref[...]
    o_a1[...] = a0 * n_ref[...]
    o_h1[...] = _bmm(x, w1_ref[...]) + b1_ref[...]
    q = _bmm(jnp.maximum(a0, 0.0), dw_ref[...]) + b2_ref[...]
    o_q[...] = jnp.maximum(q, 0.0)


_stage1 = pl.pallas_call(
    _stage1_body,
    grid=(_GRID,),
    in_specs=[_row_spec, _col_spec, _w_spec, _b_spec, _w_spec, _b_spec,
              _w_spec, _b_spec],
    out_specs=[_row_spec, _row_spec, _row_spec],
    out_shape=[jax.ShapeDtypeStruct((NN, DD), jnp.float32)] * 3,
)


def _stage2_body(a1_ref, p0_ref, p1_ref, n_ref, w_ref, b_ref, o_ref):
    x1 = a1_ref[...] + (p0_ref[...] + p1_ref[...]) * (1.0 - n_ref[...])
    o_ref[...] = _bmm(x1, w_ref[...]) + b_ref[...]


_stage2 = pl.pallas_call(
    _stage2_body,
    grid=(_GRID,),
    in_specs=[_row_spec, _row_spec, _row_spec, _col_spec, _w_spec, _b_spec],
    out_specs=_row_spec,
    out_shape=jax.ShapeDtypeStruct((NN, DD), jnp.float32),
)


def _mm2_body(p0_ref, p1_ref, w_ref, b_ref, o_ref):
    o_ref[...] = _bmm(p0_ref[...] + p1_ref[...], w_ref[...]) + b_ref[...]


_mm2 = pl.pallas_call(
    _mm2_body,
    grid=(_GRID,),
    in_specs=[_row_spec, _row_spec, _w_spec, _b_spec],
    out_specs=_row_spec,
    out_shape=jax.ShapeDtypeStruct((NN, DD), jnp.float32),
)


def _head_body(o1_ref, o2_ref, c1b_ref, c2b_ref, cls_ref, cb_ref,
               am_ref, sel_ref):
    s1 = o1_ref[...] + c1b_ref[0, 0]
    s2 = o2_ref[...] + c2b_ref[0, 0]
    sel = jnp.concatenate([s2, s1], axis=1)
    sel_ref[...] = sel
    pre = _bmm(sel, cls_ref[...]) + cb_ref[...]
    mx = jnp.max(pre, axis=1, keepdims=True)
    io = lax.broadcasted_iota(jnp.int32, pre.shape, 1)
    am_ref[...] = jnp.min(jnp.where(pre == mx, io, NCLS), axis=1,
                          keepdims=True)


_head = pl.pallas_call(
    _head_body,
    in_specs=[
        pl.BlockSpec(memory_space=pltpu.VMEM),
        pl.BlockSpec(memory_space=pltpu.VMEM),
        pl.BlockSpec(memory_space=pltpu.SMEM),
        pl.BlockSpec(memory_space=pltpu.SMEM),
        pl.BlockSpec(memory_space=pltpu.VMEM),
        pl.BlockSpec(memory_space=pltpu.VMEM),
    ],
    out_specs=[pl.BlockSpec(memory_space=pltpu.VMEM),
               pl.BlockSpec(memory_space=pltpu.VMEM)],
    out_shape=[jax.ShapeDtypeStruct((BB, 1), jnp.int32),
               jax.ShapeDtypeStruct((BB, 2 * LL), jnp.float32)],
)


def kernel(A1_tensor, adj_indices, adj_values, batch_idx, n, Lin1, Lin1_bias,
           W1, b1, W2, b2, W3, b3, weight2, bias2, conv1_w, conv1_b,
           conv2_w, conv2_b, classifier, classifier_bias):
    A1 = A1_tensor[:, 1:]
    perm = jnp.argsort(adj_indices[0], stable=True)
    src = jnp.take(adj_indices[1], perm)
    dst_w = jnp.take(adj_indices[0], perm).reshape(NWORK, 1, EPW)
    pad_f = jnp.full((NWORK, 1, 8), -1, jnp.int32)
    pad_b = jnp.full((NWORK, 1, 16), -1, jnp.int32)
    dst_s = jnp.concatenate([pad_f, dst_w, pad_b], axis=2)
    vals_s = jnp.take(adj_values, perm)
    bi3d = batch_idx
    dw = weight2.T
    lb = Lin1_bias.reshape(1, DD)
    b1r = b1.reshape(1, DD)
    b2r = b2.reshape(1, DD)
    b3r = b3.reshape(1, DD)
    bias2r = bias2.reshape(1, LL)
    cbr = classifier_bias.reshape(1, NCLS)

    a1, h1, q_raw = _stage1(A1, n, Lin1, lb, W1, b1r, dw, bias2r)
    q_tab = _round_bf16(q_raw)
    p = _spmm_call(src, dst_s, vals_s, h1)
    h2 = _stage2(a1, p[0, :NN], p[1, :NN], n, W2, b2r)
    q = _spmm_call(src, dst_s, vals_s, h2)
    h3 = _mm2(q[0, :NN], q[1, :NN], W3, b3r)
    r = _spmm_call(src, dst_s, vals_s, h3)
    p_tab = _round_bf16(_mm2(r[0, :NN], r[1, :NN], dw, bias2r))
    c1w = _round_bf16(conv1_w)
    c2w = _round_bf16(conv2_w)
    out1, out2 = _gred_call(p_tab, q_tab, bi3d, c1w, c2w)
    am, sel = _head(out1.reshape(BB, LL), out2.reshape(BB, LL),
                    conv1_b.reshape(1, 1), conv2_b.reshape(1, 1),
                    classifier, cbr)
    return am.reshape(BB), sel
